# baseline (speedup 1.0000x reference)
import functools

import jax
import jax.numpy as jnp
from jax import lax
from jax.experimental import pallas as pl
from jax.experimental.pallas import tpu as pltpu
from jax.experimental.pallas import tpu_sc as plsc

_NUM_CORES = 2
_NUM_SUBCORES = 16
_NUM_WORKERS = _NUM_CORES * _NUM_SUBCORES
_LANES = 16
_CHUNK = 128
_BLK = 512


@functools.lru_cache(maxsize=None)
def _make_relayout(n: int, dim: int):
    n_blocks = n // _BLK
    tail = n - n_blocks * _BLK
    mesh = plsc.VectorSubcoreMesh(core_axis_name="c", subcore_axis_name="s")

    @functools.partial(
        pl.kernel,
        mesh=mesh,
        out_type=jax.ShapeDtypeStruct((n, 128), jnp.float32),
        compiler_params=pltpu.CompilerParams(needs_layout_passes=False),
        scratch_types=[
            pltpu.VMEM((dim, _BLK), jnp.float32),
            pltpu.VMEM((_BLK, 128), jnp.float32),
            pltpu.SemaphoreType.DMA,
        ],
    )
    def relayout_kernel(table_t, out_hbm, stage_v, rows_v, sem):
        wid = lax.axis_index("s") * _NUM_CORES + lax.axis_index("c")
        iota = lax.iota(jnp.int32, _LANES)
        per_w = (n_blocks + _NUM_WORKERS - 1) // _NUM_WORKERS

        def do_block(c0, nrows):
            copies = [
                pltpu.async_copy(
                    table_t.at[pl.ds(8 * p, 8), pl.ds(c0, nrows)],
                    stage_v.at[pl.ds(8 * p, 8), pl.ds(0, nrows)],
                    sem,
                )
                for p in range(dim // 8)
            ]
            for c in copies:
                c.wait()

            def tr_body(r, _):
                row = jnp.zeros((_LANES,), jnp.int32) + r
                for cw in range(dim // _LANES):
                    vals = plsc.load_gather(stage_v, [cw * _LANES + iota, row])
                    rows_v[r, pl.ds(cw * _LANES, _LANES)] = vals
                return ()

            lax.fori_loop(0, nrows, tr_body, (), unroll=False)
            pltpu.sync_copy(
                rows_v.at[pl.ds(0, nrows)], out_hbm.at[pl.ds(c0, nrows)]
            )

        def blk_body(i, _):
            blk = wid + _NUM_WORKERS * i

            @pl.when(blk < n_blocks)
            def _():
                do_block(pl.multiple_of(blk * _BLK, 128), _BLK)

            return ()

        lax.fori_loop(0, per_w, blk_body, (), unroll=False)

    return relayout_kernel


@functools.lru_cache(maxsize=None)
def _make_gather(batch: int, main_n: int, n_tail: int):
    b_per_w = batch // _NUM_WORKERS
    n_chunks = b_per_w // _CHUNK
    mesh = plsc.VectorSubcoreMesh(core_axis_name="c", subcore_axis_name="s")

    @functools.partial(
        pl.kernel,
        mesh=mesh,
        out_type=jax.ShapeDtypeStruct((batch, 128), jnp.float32),
        compiler_params=pltpu.CompilerParams(needs_layout_passes=False),
        scratch_types=[
            pltpu.VMEM((b_per_w,), jnp.int32),
            pltpu.VMEM((n_chunks, _CHUNK), jnp.int32),
            pltpu.VMEM((n_tail, 128), jnp.float32),
            pltpu.VMEM((b_per_w, 128), jnp.float32),
            pltpu.SemaphoreType.DMA,
        ],
    )
    def gather_kernel(table_hbm, idx_hbm, tail_hbm, out_hbm, idx_v, cidx_v,
                      tail_v, rows_v, sem):
        wid = lax.axis_index("s") * _NUM_CORES + lax.axis_index("c")
        base = wid * b_per_w
        iota = lax.iota(jnp.int32, _LANES)
        pltpu.sync_copy(idx_hbm.at[pl.ds(base, b_per_w)], idx_v)
        pltpu.sync_copy(tail_hbm, tail_v)

        def prep_body(k, _):
            v = idx_v[pl.ds(k * _LANES, _LANES)]
            j = k // (_CHUNK // _LANES)
            o = (k % (_CHUNK // _LANES)) * _LANES
            cidx_v[j, pl.ds(o, _LANES)] = jnp.minimum(v, main_n - 1)
            return ()

        lax.fori_loop(0, b_per_w // _LANES, prep_body, (), unroll=False)

        copies = []
        for j in range(n_chunks):
            copies.append(
                pltpu.async_copy(
                    table_hbm.at[cidx_v.at[j]],
                    rows_v.at[pl.ds(j * _CHUNK, _CHUNK)],
                    sem,
                )
            )
        for c in copies:
            c.wait()

        def fix_body(k, _):
            v = idx_v[pl.ds(k * _LANES, _LANES)]
            has_tail = jnp.sum((v >= main_n).astype(jnp.int32)) > 0

            @pl.when(has_tail)
            def _():
                def one(b, _):
                    row = jnp.zeros((_LANES,), jnp.int32) + (k * _LANES + b)
                    vb = plsc.load_gather(idx_v, [row])
                    tr = vb - main_n

                    @pl.when(jnp.sum((tr >= 0).astype(jnp.int32)) > 0)
                    def _():
                        trc = jnp.maximum(tr, 0)
                        for c0 in range(128 // _LANES):
                            vals = plsc.load_gather(
                                tail_v, [trc, c0 * _LANES + iota]
                            )
                            rows_v[k * _LANES + b,
                                   pl.ds(c0 * _LANES, _LANES)] = vals
                    return ()

                lax.fori_loop(0, _LANES, one, (), unroll=False)
            return ()

        lax.fori_loop(0, b_per_w // _LANES, fix_body, (), unroll=False)
        pltpu.sync_copy(rows_v, out_hbm.at[pl.ds(base, b_per_w)])

    return gather_kernel


def kernel(input_x, Emb):
    batch = input_x.shape[1]
    n, dim = Emb.shape
    main_n = (n // _BLK) * _BLK
    table = _make_relayout(n, dim)(Emb.T)
    tail = jnp.pad(Emb[main_n:], ((0, 0), (0, 128 - dim)))
    idx = input_x.reshape(batch)
    out = _make_gather(batch, main_n, n - main_n)(table, idx, tail)
    return out[:, :dim]


# speedup vs baseline: 3.2708x; 3.2708x over previous
import functools

import jax
import jax.numpy as jnp
from jax import lax
from jax.experimental import pallas as pl
from jax.experimental.pallas import tpu as pltpu
from jax.experimental.pallas import tpu_sc as plsc

_NUM_CORES = 2
_NUM_SUBCORES = 16
_NUM_WORKERS = _NUM_CORES * _NUM_SUBCORES
_CHUNK = 128


@functools.lru_cache(maxsize=None)
def _make_gather(batch: int, dim_padded: int):
    b_per_w = batch // _NUM_WORKERS
    n_chunks = b_per_w // _CHUNK
    mesh = plsc.VectorSubcoreMesh(core_axis_name="c", subcore_axis_name="s")

    @functools.partial(
        pl.kernel,
        mesh=mesh,
        out_type=jax.ShapeDtypeStruct((batch, dim_padded), jnp.float32),
        scratch_types=[
            pltpu.VMEM((n_chunks, _CHUNK), jnp.int32),
            pltpu.VMEM((b_per_w, dim_padded), jnp.float32),
            pltpu.SemaphoreType.DMA,
        ],
    )
    def gather_kernel(table_hbm, idx_hbm, out_hbm, idx_v, rows_v, sem):
        wid = lax.axis_index("s") * _NUM_CORES + lax.axis_index("c")
        base = wid * b_per_w
        pltpu.sync_copy(idx_hbm.at[wid], idx_v)
        copies = []
        for j in range(n_chunks):
            copies.append(
                pltpu.async_copy(
                    table_hbm.at[idx_v.at[j]],
                    rows_v.at[pl.ds(j * _CHUNK, _CHUNK)],
                    sem,
                )
            )
        for c in copies:
            c.wait()
        pltpu.sync_copy(rows_v, out_hbm.at[pl.ds(base, b_per_w)])

    return gather_kernel


def kernel(input_x, Emb):
    batch = input_x.shape[1]
    dim = Emb.shape[1]
    table = jnp.pad(Emb, ((0, 0), (0, 128 - dim)))
    idx = input_x.reshape(_NUM_WORKERS, batch // _NUM_WORKERS // _CHUNK, _CHUNK)
    out = _make_gather(batch, 128)(table, idx)
    return out[:, :dim]
